# baseline (speedup 1.0000x reference)
import functools

import jax
import jax.numpy as jnp
from jax import lax
from jax.experimental import pallas as pl
from jax.experimental.pallas import tpu as pltpu
from jax.experimental.pallas import tpu_sc as plsc

B, N, M = 8, 2048, 2048
NW = 32
WPB = NW // B
R = N // WPB
L = 16
BIG = 3.0e38


def _make_sc_kernel():
    mesh = plsc.VectorSubcoreMesh(core_axis_name="c", subcore_axis_name="s")
    f32 = jnp.float32

    @functools.partial(
        pl.kernel,
        mesh=mesh,
        out_type=[
            jax.ShapeDtypeStruct((B * N,), f32),
            jax.ShapeDtypeStruct((NW * M,), f32),
        ],
        scratch_types=[
            pltpu.VMEM((R,), f32),
            pltpu.VMEM((R,), f32),
            pltpu.VMEM((R,), f32),
            pltpu.VMEM((M,), f32),
            pltpu.VMEM((M,), f32),
            pltpu.VMEM((M,), f32),
            pltpu.VMEM((M,), f32),
            pltpu.VMEM((R,), f32),
        ],
    )
    def sck(pxh, pyh, pzh, txh, tyh, tzh, rowmin_hbm, colmin_hbm,
            px_v, py_v, pz_v, tx_v, ty_v, tz_v, cm_v, rm_v):
        c = lax.axis_index("c")
        s = lax.axis_index("s")
        wid = c * 16 + s
        b = wid // WPB
        base = (wid % WPB) * R
        pltpu.sync_copy(pxh.at[pl.ds(b * N + base, R)], px_v)
        pltpu.sync_copy(pyh.at[pl.ds(b * N + base, R)], py_v)
        pltpu.sync_copy(pzh.at[pl.ds(b * N + base, R)], pz_v)
        pltpu.sync_copy(txh.at[pl.ds(b * M, M)], tx_v)
        pltpu.sync_copy(tyh.at[pl.ds(b * M, M)], ty_v)
        pltpu.sync_copy(tzh.at[pl.ds(b * M, M)], tz_v)

        def init_cm(jc, _):
            cm_v[pl.ds(jc * L, L)] = jnp.full((L,), BIG, f32)
            return 0

        lax.fori_loop(0, M // L, init_cm, 0)

        lane = lax.broadcasted_iota(jnp.int32, (L,), 0)

        def vmin_all(v):
            for sh in (8, 4, 2, 1):
                idx = jnp.bitwise_and(lane + sh, L - 1)
                v = jnp.minimum(v, v[idx])
            return v

        def chunk_body(ic, _):
            pvx = px_v[pl.ds(ic * L, L)]
            pvy = py_v[pl.ds(ic * L, L)]
            pvz = pz_v[pl.ds(ic * L, L)]

            def j_body(jc, rowaccs):
                off = jc * L
                txv = tx_v[pl.ds(off, L)]
                tyv = ty_v[pl.ds(off, L)]
                tzv = tz_v[pl.ds(off, L)]
                colacc = cm_v[pl.ds(off, L)]
                new_accs = []
                for k in range(L):
                    dx = txv - pvx[k]
                    dy = tyv - pvy[k]
                    dz = tzv - pvz[k]
                    d = dx * dx + dy * dy + dz * dz
                    colacc = jnp.minimum(colacc, d)
                    new_accs.append(jnp.minimum(rowaccs[k], d))
                cm_v[pl.ds(off, L)] = colacc
                return tuple(new_accs)

            rowaccs = lax.fori_loop(
                0, M // L, j_body,
                tuple(jnp.full((L,), BIG, f32) for _ in range(L)))
            rowvec = jnp.full((L,), BIG, f32)
            for k in range(L):
                rowvec = jnp.where(lane == k, vmin_all(rowaccs[k]), rowvec)
            rm_v[pl.ds(ic * L, L)] = rowvec
            return 0

        lax.fori_loop(0, R // L, chunk_body, 0)

        pltpu.sync_copy(rm_v, rowmin_hbm.at[pl.ds(b * N + base, R)])
        pltpu.sync_copy(cm_v, colmin_hbm.at[pl.ds(wid * M, M)])

    return sck


_sck = _make_sc_kernel()


@jax.jit
def kernel(pred, target):
    coords_p = [pred[:, :, c].reshape(-1) for c in range(3)]
    coords_t = [target[:, :, c].reshape(-1) for c in range(3)]
    rowmin, colmin_parts = _sck(*coords_p, *coords_t)
    colmin = jnp.min(colmin_parts.reshape(B, WPB, M), axis=1)
    return jnp.mean(rowmin) + jnp.mean(colmin)


# speedup vs baseline: 5.2900x; 5.2900x over previous
import functools

import jax
import jax.numpy as jnp
from jax.experimental import pallas as pl


TILE_I = 2048


def _chamfer_body(pred_ref, tgt_ref, minp_ref, mint_ref):
    it = pl.program_id(1)
    px = pred_ref[0, :, 0:1]
    py = pred_ref[0, :, 1:2]
    pz = pred_ref[0, :, 2:3]
    tx = tgt_ref[0, 0:1, :]
    ty = tgt_ref[0, 1:2, :]
    tz = tgt_ref[0, 2:3, :]
    mtx, mty, mtz = -2.0 * tx, -2.0 * ty, -2.0 * tz
    tn = tx * tx + ty * ty + tz * tz
    pn = px * px + py * py + pz * pz
    f = (tn + px * mtx) + (py * mty + pz * mtz)
    minp_ref[0, 0, :] = jnp.min(f, axis=1) + pn[:, 0]
    colmin = jnp.min(f + pn, axis=0, keepdims=True)

    @pl.when(it == 0)
    def _init():
        mint_ref[0] = colmin

    @pl.when(it != 0)
    def _acc():
        mint_ref[0] = jnp.minimum(mint_ref[0], colmin)


@functools.partial(jax.jit, static_argnames=("interpret",))
def kernel(pred, target, interpret=False):
    B, N, _ = pred.shape
    M = target.shape[1]
    tgt_t = jnp.swapaxes(target, 1, 2)
    grid = (B, N // TILE_I)
    minp, mint = pl.pallas_call(
        _chamfer_body,
        grid=grid,
        in_specs=[
            pl.BlockSpec((1, TILE_I, 3), lambda b, it: (b, it, 0)),
            pl.BlockSpec((1, 3, M), lambda b, it: (b, 0, 0)),
        ],
        out_specs=[
            pl.BlockSpec((1, 1, TILE_I),
                         lambda b, it: (b * (N // TILE_I) + it, 0, 0)),
            pl.BlockSpec((1, 1, M), lambda b, it: (b, 0, 0)),
        ],
        out_shape=[
            jax.ShapeDtypeStruct((B * (N // TILE_I), 1, TILE_I), jnp.float32),
            jax.ShapeDtypeStruct((B, 1, M), jnp.float32),
        ],
        interpret=interpret,
    )(pred, tgt_t)
    return jnp.mean(minp) + jnp.mean(mint)


# speedup vs baseline: 5.3409x; 1.0096x over previous
import functools

import jax
import jax.numpy as jnp
from jax.experimental import pallas as pl


TILE_I = 2048


def _chamfer_body(pred_ref, tgt_ref, minp_ref, mint_ref):
    it = pl.program_id(1)
    px = pred_ref[0, :, 0:1]
    py = pred_ref[0, :, 1:2]
    pz = pred_ref[0, :, 2:3]
    tx = tgt_ref[0, 0:1, :]
    ty = tgt_ref[0, 1:2, :]
    tz = tgt_ref[0, 2:3, :]
    d = (px - tx) ** 2 + (py - ty) ** 2 + (pz - tz) ** 2
    minp_ref[0, 0, :] = jnp.min(d, axis=1)
    colmin = jnp.min(d, axis=0, keepdims=True)

    @pl.when(it == 0)
    def _init():
        mint_ref[0] = colmin

    @pl.when(it != 0)
    def _acc():
        mint_ref[0] = jnp.minimum(mint_ref[0], colmin)


@functools.partial(jax.jit, static_argnames=("interpret",))
def kernel(pred, target, interpret=False):
    B, N, _ = pred.shape
    M = target.shape[1]
    tgt_t = jnp.swapaxes(target, 1, 2)
    grid = (B, N // TILE_I)
    minp, mint = pl.pallas_call(
        _chamfer_body,
        grid=grid,
        in_specs=[
            pl.BlockSpec((1, TILE_I, 3), lambda b, it: (b, it, 0)),
            pl.BlockSpec((1, 3, M), lambda b, it: (b, 0, 0)),
        ],
        out_specs=[
            pl.BlockSpec((1, 1, TILE_I),
                         lambda b, it: (b * (N // TILE_I) + it, 0, 0)),
            pl.BlockSpec((1, 1, M), lambda b, it: (b, 0, 0)),
        ],
        out_shape=[
            jax.ShapeDtypeStruct((B * (N // TILE_I), 1, TILE_I), jnp.float32),
            jax.ShapeDtypeStruct((B, 1, M), jnp.float32),
        ],
        interpret=interpret,
    )(pred, tgt_t)
    return jnp.mean(minp) + jnp.mean(mint)


# speedup vs baseline: 5.3429x; 1.0004x over previous
import jax
import jax.numpy as jnp
from jax.experimental import pallas as pl


TILE_I = 2048


def _chamfer_body(pred_ref, tgt_ref, minp_ref, mint_ref):
    it = pl.program_id(1)
    px = pred_ref[0, :, 0:1]
    py = pred_ref[0, :, 1:2]
    pz = pred_ref[0, :, 2:3]
    tx = tgt_ref[0, 0:1, :]
    ty = tgt_ref[0, 1:2, :]
    tz = tgt_ref[0, 2:3, :]
    d = (px - tx) ** 2 + (py - ty) ** 2 + (pz - tz) ** 2
    minp_ref[0, 0, :] = jnp.min(d, axis=1)
    colmin = jnp.min(d, axis=0, keepdims=True)

    @pl.when(it == 0)
    def _init():
        mint_ref[0] = colmin

    @pl.when(it != 0)
    def _acc():
        mint_ref[0] = jnp.minimum(mint_ref[0], colmin)


@jax.jit
def kernel(pred, target):
    B, N, _ = pred.shape
    M = target.shape[1]
    tgt_t = jnp.swapaxes(target, 1, 2)
    grid = (B, N // TILE_I)
    minp, mint = pl.pallas_call(
        _chamfer_body,
        grid=grid,
        in_specs=[
            pl.BlockSpec((1, TILE_I, 3), lambda b, it: (b, it, 0)),
            pl.BlockSpec((1, 3, M), lambda b, it: (b, 0, 0)),
        ],
        out_specs=[
            pl.BlockSpec((1, 1, TILE_I),
                         lambda b, it: (b * (N // TILE_I) + it, 0, 0)),
            pl.BlockSpec((1, 1, M), lambda b, it: (b, 0, 0)),
        ],
        out_shape=[
            jax.ShapeDtypeStruct((B * (N // TILE_I), 1, TILE_I), jnp.float32),
            jax.ShapeDtypeStruct((B, 1, M), jnp.float32),
        ],
    )(pred, tgt_t)
    return jnp.mean(minp) + jnp.mean(mint)
